# baseline (speedup 1.0000x reference)
import functools

import jax
import jax.numpy as jnp
from jax import lax
from jax.experimental import pallas as pl
from jax.experimental.pallas import tpu as pltpu
from jax.experimental.pallas import tpu_sc as plsc

_N_DST = 10000
_N_EDGES = 320000
_D = 128

_NC = 2
_NS = 16
_NW = _NC * _NS

_EDGES_PER_TILE = _N_EDGES // _NW
_B = 80
_NBLK = _EDGES_PER_TILE // _B
_B_TAIL = _EDGES_PER_TILE - _NBLK * _B

_CH = 80
_NCHUNKS = _N_DST // _CH


def _sc_segment_sum(src_emb, dst_ids):
    mesh = plsc.VectorSubcoreMesh(core_axis_name="c", subcore_axis_name="s")

    @functools.partial(
        pl.kernel,
        mesh=mesh,
        out_type=jax.ShapeDtypeStruct((_NC, _N_DST, _D), jnp.float32),
        scratch_types=[
            pltpu.VMEM((_B, _D), jnp.float32),
            pltpu.VMEM((_B, _D), jnp.float32),
            pltpu.VMEM((_B,), jnp.int32),
            pltpu.VMEM((_B,), jnp.int32),
            pltpu.VMEM((max(_B_TAIL, 8), _D), jnp.float32),
            pltpu.VMEM((max(_B_TAIL, 8),), jnp.int32),
            pltpu.VMEM((_CH, _D), jnp.float32),
            pltpu.VMEM_SHARED((_N_DST, _D), jnp.float32),
            pltpu.SemaphoreType.DMA,
            pltpu.SemaphoreType.DMA,
        ],
    )
    def body(src_hbm, ids_hbm, out_hbm, rows_a, rows_b, ids_a, ids_b,
             rows_t, ids_t, tmp_v, acc_sh, sem_a, sem_b):
        c = lax.axis_index("c")
        s = lax.axis_index("s")
        w = s * _NC + c

        z16 = jnp.zeros((16,), jnp.float32)

        def zero_row(r, carry):
            for j in range(_D // 16):
                tmp_v[r, pl.ds(j * 16, 16)] = z16
            return carry

        lax.fori_loop(0, _CH, zero_row, 0)

        n_my_chunks = (_NCHUNKS + _NS - 1 - s) // _NS

        def init_chunk(k, carry):
            r0 = pl.multiple_of((s + k * _NS) * _CH, 8)
            pltpu.sync_copy(tmp_v, acc_sh.at[pl.ds(r0, _CH)])
            return carry

        lax.fori_loop(0, n_my_chunks, init_chunk, 0)
        plsc.subcore_barrier()

        def _base(b):
            return pl.multiple_of(w * _EDGES_PER_TILE + b * _B, _B)

        def start(b, rows_v, ids_v, sem):
            base = _base(b)
            pltpu.async_copy(src_hbm.at[pl.ds(base, _B)], rows_v, sem)
            pltpu.async_copy(ids_hbm.at[pl.ds(base, _B)], ids_v, sem)

        def dma_wait(b, rows_v, ids_v, sem):
            base = _base(b)
            pltpu.make_async_copy(src_hbm.at[pl.ds(base, _B)], rows_v, sem).wait()
            pltpu.make_async_copy(ids_hbm.at[pl.ds(base, _B)], ids_v, sem).wait()

        start(0, rows_a, ids_a, sem_a)

        def pair(g, carry):
            start(2 * g + 1, rows_b, ids_b, sem_b)
            dma_wait(2 * g, rows_a, ids_a, sem_a)
            d_a = pltpu.async_copy(rows_a, acc_sh.at[ids_a], sem_a, add=True)
            dma_wait(2 * g + 1, rows_b, ids_b, sem_b)
            d_b = pltpu.async_copy(rows_b, acc_sh.at[ids_b], sem_b, add=True)
            d_a.wait()
            start(2 * g + 2, rows_a, ids_a, sem_a)
            d_b.wait()
            return carry

        lax.fori_loop(0, (_NBLK - 1) // 2, pair, 0)
        dma_wait(_NBLK - 1, rows_a, ids_a, sem_a)
        pltpu.sync_copy(rows_a, acc_sh.at[ids_a], add=True)

        plsc.subcore_barrier()

        def out_chunk(k, carry):
            r0 = pl.multiple_of((s + k * _NS) * _CH, 8)
            pltpu.sync_copy(acc_sh.at[pl.ds(r0, _CH)], tmp_v)
            pltpu.sync_copy(tmp_v, out_hbm.at[c, pl.ds(r0, _CH)])
            return carry

        lax.fori_loop(0, n_my_chunks, out_chunk, 0)

    return body(src_emb, dst_ids)


_R_BLK = 2000


def _combine_body(parts_ref, tail_ref, o_ref):
    o_ref[...] = parts_ref[0] + parts_ref[1] + tail_ref[...]


def _combine(parts, tail):
    return pl.pallas_call(
        _combine_body,
        grid=(_N_DST // _R_BLK,),
        in_specs=[
            pl.BlockSpec((_NC, _R_BLK, _D), lambda i: (0, i, 0)),
            pl.BlockSpec((_R_BLK, _D), lambda i: (i, 0)),
        ],
        out_specs=pl.BlockSpec((_R_BLK, _D), lambda i: (i, 0)),
        out_shape=jax.ShapeDtypeStruct((_N_DST, _D), jnp.float32),
    )(parts, tail)


def kernel(src_emb, src_emb_in, dst_ids):
    del src_emb_in
    parts = _sc_segment_sum(src_emb, dst_ids.astype(jnp.int32))
    tail = lax.slice_in_dim(src_emb, _N_EDGES, _N_EDGES + _N_DST, axis=0)
    return _combine(parts, tail)


# speedup vs baseline: 1.2884x; 1.2884x over previous
import functools

import jax
import jax.numpy as jnp
from jax import lax
from jax.experimental import pallas as pl
from jax.experimental.pallas import tpu as pltpu
from jax.experimental.pallas import tpu_sc as plsc

_N_DST = 10000
_N_EDGES = 320000
_D = 128

_NC = 2
_NS = 16
_NW = _NC * _NS

_EDGES_PER_TILE = _N_EDGES // _NW
_B = 80
_NBLK = _EDGES_PER_TILE // _B
_B_TAIL = _EDGES_PER_TILE - _NBLK * _B

_CH = 80
_NCHUNKS = _N_DST // _CH


def _sc_segment_sum(src_emb, dst_ids):
    mesh = plsc.VectorSubcoreMesh(core_axis_name="c", subcore_axis_name="s")

    @functools.partial(
        pl.kernel,
        mesh=mesh,
        out_type=jax.ShapeDtypeStruct((_NC, _N_DST, _D), jnp.float32),
        scratch_types=[
            pltpu.VMEM((_B, _D), jnp.float32),
            pltpu.VMEM((_B, _D), jnp.float32),
            pltpu.VMEM((_B,), jnp.int32),
            pltpu.VMEM((_B,), jnp.int32),
            pltpu.VMEM((max(_B_TAIL, 8), _D), jnp.float32),
            pltpu.VMEM((max(_B_TAIL, 8),), jnp.int32),
            pltpu.VMEM((_CH, _D), jnp.float32),
            pltpu.VMEM_SHARED((_N_DST, _D), jnp.float32),
            pltpu.SemaphoreType.DMA,
            pltpu.SemaphoreType.DMA,
        ],
    )
    def body(src_hbm, ids_hbm, out_hbm, rows_a, rows_b, ids_a, ids_b,
             rows_t, ids_t, tmp_v, acc_sh, sem_a, sem_b):
        c = lax.axis_index("c")
        s = lax.axis_index("s")
        w = s * _NC + c

        z16 = jnp.zeros((16,), jnp.float32)

        def zero_row(r, carry):
            for j in range(_D // 16):
                tmp_v[r, pl.ds(j * 16, 16)] = z16
            return carry

        lax.fori_loop(0, _CH, zero_row, 0)

        n_my_chunks = (_NCHUNKS + _NS - 1 - s) // _NS

        def init_chunk(k, carry):
            r0 = pl.multiple_of((s + k * _NS) * _CH, 8)
            pltpu.sync_copy(tmp_v, acc_sh.at[pl.ds(r0, _CH)])
            return carry

        lax.fori_loop(0, n_my_chunks, init_chunk, 0)
        plsc.subcore_barrier()

        def _base(b):
            return pl.multiple_of(w * _EDGES_PER_TILE + b * _B, _B)

        def start(b, rows_v, ids_v, sem):
            base = _base(b)
            pltpu.async_copy(src_hbm.at[pl.ds(base, _B)], rows_v, sem)
            pltpu.async_copy(ids_hbm.at[pl.ds(base, _B)], ids_v, sem)

        def dma_wait(b, rows_v, ids_v, sem):
            base = _base(b)
            pltpu.make_async_copy(src_hbm.at[pl.ds(base, _B)], rows_v, sem).wait()
            pltpu.make_async_copy(ids_hbm.at[pl.ds(base, _B)], ids_v, sem).wait()

        start(0, rows_a, ids_a, sem_a)

        def pair(g, carry):
            start(2 * g + 1, rows_b, ids_b, sem_b)
            dma_wait(2 * g, rows_a, ids_a, sem_a)
            pltpu.sync_copy(rows_a, acc_sh.at[ids_a], add=True)
            start(2 * g + 2, rows_a, ids_a, sem_a)
            dma_wait(2 * g + 1, rows_b, ids_b, sem_b)
            pltpu.sync_copy(rows_b, acc_sh.at[ids_b], add=True)
            return carry

        lax.fori_loop(0, (_NBLK - 1) // 2, pair, 0)
        dma_wait(_NBLK - 1, rows_a, ids_a, sem_a)
        pltpu.sync_copy(rows_a, acc_sh.at[ids_a], add=True)

        plsc.subcore_barrier()

        def out_chunk(k, carry):
            r0 = pl.multiple_of((s + k * _NS) * _CH, 8)
            pltpu.sync_copy(acc_sh.at[pl.ds(r0, _CH)], out_hbm.at[c, pl.ds(r0, _CH)])
            return carry

        lax.fori_loop(0, n_my_chunks, out_chunk, 0)

    return body(src_emb, dst_ids)


_R_BLK = 2000


def _combine_body(parts_ref, tail_ref, o_ref):
    o_ref[...] = parts_ref[0] + parts_ref[1] + tail_ref[...]


def _combine(parts, tail):
    return pl.pallas_call(
        _combine_body,
        grid=(_N_DST // _R_BLK,),
        in_specs=[
            pl.BlockSpec((_NC, _R_BLK, _D), lambda i: (0, i, 0)),
            pl.BlockSpec((_R_BLK, _D), lambda i: (i, 0)),
        ],
        out_specs=pl.BlockSpec((_R_BLK, _D), lambda i: (i, 0)),
        out_shape=jax.ShapeDtypeStruct((_N_DST, _D), jnp.float32),
    )(parts, tail)


def kernel(src_emb, src_emb_in, dst_ids):
    del src_emb_in
    parts = _sc_segment_sum(src_emb, dst_ids.astype(jnp.int32))
    tail = lax.slice_in_dim(src_emb, _N_EDGES, _N_EDGES + _N_DST, axis=0)
    return _combine(parts, tail)


# speedup vs baseline: 1.3012x; 1.0099x over previous
import functools

import jax
import jax.numpy as jnp
from jax import lax
from jax.experimental import pallas as pl
from jax.experimental.pallas import tpu as pltpu
from jax.experimental.pallas import tpu_sc as plsc

_N_DST = 10000
_N_EDGES = 320000
_D = 128

_NC = 2
_NS = 16
_NW = _NC * _NS

_EDGES_PER_TILE = _N_EDGES // _NW
_B = 80
_NBLK = _EDGES_PER_TILE // _B
_B_TAIL = _EDGES_PER_TILE - _NBLK * _B

_CH = 80
_NCHUNKS = _N_DST // _CH


def _sc_segment_sum(src_emb, dst_ids):
    mesh = plsc.VectorSubcoreMesh(core_axis_name="c", subcore_axis_name="s")

    @functools.partial(
        pl.kernel,
        mesh=mesh,
        out_type=jax.ShapeDtypeStruct((_NC, _N_DST, _D), jnp.float32),
        scratch_types=[
            pltpu.VMEM((_B, _D), jnp.float32),
            pltpu.VMEM((_B, _D), jnp.float32),
            pltpu.VMEM((_B,), jnp.int32),
            pltpu.VMEM((_B,), jnp.int32),
            pltpu.VMEM((_CH, _D), jnp.float32),
            pltpu.VMEM_SHARED((_N_DST, _D), jnp.float32),
            pltpu.SemaphoreType.DMA,
            pltpu.SemaphoreType.DMA,
        ],
    )
    def body(src_hbm, ids_hbm, out_hbm, rows_a, rows_b, ids_a, ids_b,
             tmp_v, acc_sh, sem_a, sem_b):
        c = lax.axis_index("c")
        s = lax.axis_index("s")
        w = s * _NC + c

        def _base(b):
            return pl.multiple_of(w * _EDGES_PER_TILE + b * _B, _B)

        def start(b, rows_v, ids_v, sem):
            base = _base(b)
            pltpu.async_copy(src_hbm.at[pl.ds(base, _B)], rows_v, sem)
            pltpu.async_copy(ids_hbm.at[pl.ds(base, _B)], ids_v, sem)

        def dma_wait(b, rows_v, ids_v, sem):
            base = _base(b)
            pltpu.make_async_copy(src_hbm.at[pl.ds(base, _B)], rows_v, sem).wait()
            pltpu.make_async_copy(ids_hbm.at[pl.ds(base, _B)], ids_v, sem).wait()

        start(0, rows_a, ids_a, sem_a)
        start(1, rows_b, ids_b, sem_b)

        z16 = jnp.zeros((16,), jnp.float32)

        def zero_row(r, carry):
            for j in range(_D // 16):
                tmp_v[r, pl.ds(j * 16, 16)] = z16
            return carry

        lax.fori_loop(0, _CH, zero_row, 0)

        n_my_chunks = (_NCHUNKS + _NS - 1 - s) // _NS

        def init_chunk(k, carry):
            r0 = pl.multiple_of((s + k * _NS) * _CH, 8)
            pltpu.sync_copy(tmp_v, acc_sh.at[pl.ds(r0, _CH)])
            return carry

        lax.fori_loop(0, n_my_chunks, init_chunk, 0)
        plsc.subcore_barrier()

        def pair(g, carry):
            dma_wait(2 * g, rows_a, ids_a, sem_a)
            pltpu.sync_copy(rows_a, acc_sh.at[ids_a], add=True)
            start(2 * g + 2, rows_a, ids_a, sem_a)
            dma_wait(2 * g + 1, rows_b, ids_b, sem_b)
            pltpu.sync_copy(rows_b, acc_sh.at[ids_b], add=True)
            start(2 * g + 3, rows_b, ids_b, sem_b)
            return carry

        lax.fori_loop(0, _NBLK // 2 - 1, pair, 0)
        dma_wait(_NBLK - 3, rows_a, ids_a, sem_a)
        pltpu.sync_copy(rows_a, acc_sh.at[ids_a], add=True)
        start(_NBLK - 1, rows_a, ids_a, sem_a)
        dma_wait(_NBLK - 2, rows_b, ids_b, sem_b)
        pltpu.sync_copy(rows_b, acc_sh.at[ids_b], add=True)
        dma_wait(_NBLK - 1, rows_a, ids_a, sem_a)
        pltpu.sync_copy(rows_a, acc_sh.at[ids_a], add=True)

        plsc.subcore_barrier()

        def out_chunk(k, carry):
            r0 = pl.multiple_of((s + k * _NS) * _CH, 8)
            pltpu.sync_copy(acc_sh.at[pl.ds(r0, _CH)], out_hbm.at[c, pl.ds(r0, _CH)])
            return carry

        lax.fori_loop(0, n_my_chunks, out_chunk, 0)

    return body(src_emb, dst_ids)


_R_BLK = 2000


def _combine_body(parts_ref, tail_ref, o_ref):
    o_ref[...] = parts_ref[0] + parts_ref[1] + tail_ref[...]


def _combine(parts, tail):
    return pl.pallas_call(
        _combine_body,
        grid=(_N_DST // _R_BLK,),
        in_specs=[
            pl.BlockSpec((_NC, _R_BLK, _D), lambda i: (0, i, 0)),
            pl.BlockSpec((_R_BLK, _D), lambda i: (i, 0)),
        ],
        out_specs=pl.BlockSpec((_R_BLK, _D), lambda i: (i, 0)),
        out_shape=jax.ShapeDtypeStruct((_N_DST, _D), jnp.float32),
    )(parts, tail)


def kernel(src_emb, src_emb_in, dst_ids):
    del src_emb_in
    parts = _sc_segment_sum(src_emb, dst_ids.astype(jnp.int32))
    tail = lax.slice_in_dim(src_emb, _N_EDGES, _N_EDGES + _N_DST, axis=0)
    return _combine(parts, tail)


# speedup vs baseline: 1.3129x; 1.0090x over previous
import functools

import jax
import jax.numpy as jnp
from jax import lax
from jax.experimental import pallas as pl
from jax.experimental.pallas import tpu as pltpu
from jax.experimental.pallas import tpu_sc as plsc

_N_DST = 10000
_N_EDGES = 320000
_D = 128

_NC = 2
_NS = 16
_NW = _NC * _NS

_EDGES_PER_TILE = _N_EDGES // _NW
_B = 80
_NBLK = _EDGES_PER_TILE // _B
_B_TAIL = _EDGES_PER_TILE - _NBLK * _B

_CH = 80
_NCHUNKS = _N_DST // _CH


def _sc_segment_sum(src_emb, dst_ids):
    mesh = plsc.VectorSubcoreMesh(core_axis_name="c", subcore_axis_name="s")

    @functools.partial(
        pl.kernel,
        mesh=mesh,
        out_type=jax.ShapeDtypeStruct((_NC, _N_DST, _D), jnp.float32),
        scratch_types=[
            pltpu.VMEM((_B, _D), jnp.float32),
            pltpu.VMEM((_B, _D), jnp.float32),
            pltpu.VMEM((_B,), jnp.int32),
            pltpu.VMEM((_B,), jnp.int32),
            pltpu.VMEM((_CH, _D), jnp.float32),
            pltpu.VMEM_SHARED((_N_DST, _D), jnp.float32),
            pltpu.SemaphoreType.DMA,
            pltpu.SemaphoreType.DMA,
        ],
    )
    def body(src_hbm, ids_hbm, out_hbm, rows_a, rows_b, ids_a, ids_b,
             tmp_v, acc_sh, sem_a, sem_b):
        c = lax.axis_index("c")
        s = lax.axis_index("s")
        w = s * _NC + c

        def _base(b):
            return pl.multiple_of(w * _EDGES_PER_TILE + b * _B, _B)

        def start(b, rows_v, ids_v, sem):
            base = _base(b)
            pltpu.async_copy(src_hbm.at[pl.ds(base, _B)], rows_v, sem)
            pltpu.async_copy(ids_hbm.at[pl.ds(base, _B)], ids_v, sem)

        def dma_wait(b, rows_v, ids_v, sem):
            base = _base(b)
            pltpu.make_async_copy(src_hbm.at[pl.ds(base, _B)], rows_v, sem).wait()
            pltpu.make_async_copy(ids_hbm.at[pl.ds(base, _B)], ids_v, sem).wait()

        start(0, rows_a, ids_a, sem_a)
        start(1, rows_b, ids_b, sem_b)

        z16 = jnp.zeros((16,), jnp.float32)

        def zero_row(r, carry):
            for j in range(_D // 16):
                tmp_v[r, pl.ds(j * 16, 16)] = z16
            return carry

        lax.fori_loop(0, _CH, zero_row, 0)

        n_my_chunks = (_NCHUNKS + _NS - 1 - s) // _NS

        def init_chunk(k, carry):
            r0 = pl.multiple_of((s + k * _NS) * _CH, 8)
            pltpu.sync_copy(tmp_v, acc_sh.at[pl.ds(r0, _CH)])
            return carry

        lax.fori_loop(0, n_my_chunks, init_chunk, 0)
        plsc.subcore_barrier()

        def pair(g, carry):
            dma_wait(2 * g, rows_a, ids_a, sem_a)
            pltpu.sync_copy(rows_a, acc_sh.at[ids_a], add=True)
            start(2 * g + 2, rows_a, ids_a, sem_a)
            dma_wait(2 * g + 1, rows_b, ids_b, sem_b)
            pltpu.sync_copy(rows_b, acc_sh.at[ids_b], add=True)
            start(2 * g + 3, rows_b, ids_b, sem_b)
            return carry

        lax.fori_loop(0, _NBLK // 2 - 1, pair, 0)
        dma_wait(_NBLK - 3, rows_a, ids_a, sem_a)
        pltpu.sync_copy(rows_a, acc_sh.at[ids_a], add=True)
        start(_NBLK - 1, rows_a, ids_a, sem_a)
        dma_wait(_NBLK - 2, rows_b, ids_b, sem_b)
        pltpu.sync_copy(rows_b, acc_sh.at[ids_b], add=True)
        dma_wait(_NBLK - 1, rows_a, ids_a, sem_a)
        pltpu.sync_copy(rows_a, acc_sh.at[ids_a], add=True)

        plsc.subcore_barrier()

        def out_chunk(k, carry):
            r0 = pl.multiple_of((s + k * _NS) * _CH, 8)
            pltpu.sync_copy(acc_sh.at[pl.ds(r0, _CH)], out_hbm.at[c, pl.ds(r0, _CH)])
            return carry

        lax.fori_loop(0, n_my_chunks, out_chunk, 0)

    return body(src_emb, dst_ids)


_R_BLK = 2000


def _combine_body(parts_ref, tail_ref, o_ref):
    o_ref[...] = parts_ref[0] + parts_ref[1] + tail_ref[...]


_TAIL_BLK0 = _N_EDGES // _R_BLK


def _combine(parts, src_emb):
    return pl.pallas_call(
        _combine_body,
        grid=(_N_DST // _R_BLK,),
        in_specs=[
            pl.BlockSpec((_NC, _R_BLK, _D), lambda i: (0, i, 0)),
            pl.BlockSpec((_R_BLK, _D), lambda i: (_TAIL_BLK0 + i, 0)),
        ],
        out_specs=pl.BlockSpec((_R_BLK, _D), lambda i: (i, 0)),
        out_shape=jax.ShapeDtypeStruct((_N_DST, _D), jnp.float32),
    )(parts, src_emb)


def kernel(src_emb, src_emb_in, dst_ids):
    del src_emb_in
    parts = _sc_segment_sum(src_emb, dst_ids.astype(jnp.int32))
    return _combine(parts, src_emb)
